# Initial kernel scaffold; baseline (speedup 1.0000x reference)
import jax
import jax.numpy as jnp
from jax.experimental import pallas as pl


def kernel(x, y):
    raise NotImplementedError("write your pallas kernel here")


# speedup vs baseline: 18173.1035x; 18173.1035x over previous
import functools

import jax
import jax.numpy as jnp
from jax import lax
from jax.experimental import pallas as pl
from jax.experimental.pallas import tpu as pltpu
from jax.experimental.pallas import tpu_sc as plsc

_NB, _NT, _NR, _NC = 8, 128, 64, 64
_CORES, _SUBCORES, _LANES = 2, 16, 16
_NW = _CORES * _SUBCORES
_PAIRS = _NB * _NR
_PPW = _PAIRS // _NW
_GROUPS = _NC // _LANES
_BIG = jnp.float32(3.0e38)
_CSQ = jnp.float32(0.001) * jnp.float32(0.001)


def _tile_body(x_hbm, y_hbm, out_hbm, xbuf, ybuf, abuf, bbuf, wstage):
    wid = lax.axis_index("s") * _CORES + lax.axis_index("c")
    lane = lax.iota(jnp.int32, _LANES)

    def do_pair(p, w_acc):
        pair = wid * _PPW + p
        b = pair // _NR
        r = pair - b * _NR
        pltpu.sync_copy(x_hbm.at[b, :, r, :], xbuf)
        pltpu.sync_copy(y_hbm.at[b, :, r, :], ybuf)

        def cum_step(t, accs):
            out = []
            for g in range(_GROUPS):
                sl = pl.ds(g * _LANES, _LANES)
                fl = pl.ds(t * _NC + g * _LANES, _LANES)
                xa = xbuf[t, sl]
                aa = accs[g] + xa * xa
                abuf[fl] = aa
                ya = ybuf[t, sl]
                bb = accs[_GROUPS + g] + ya * ya
                bbuf[fl] = bb
                out.append(aa)
                out.append(bb)
            return tuple(out[0::2]) + tuple(out[1::2])

        zeros = tuple(jnp.zeros((_LANES,), jnp.float32) for _ in range(2 * _GROUPS))
        totals = lax.fori_loop(0, _NT, cum_step, zeros)

        big = jnp.full((_LANES,), _BIG, jnp.float32)
        for g in range(_GROUPS):
            fl = pl.ds((_NT - 1) * _NC + g * _LANES, _LANES)
            abuf[fl] = big
            bbuf[fl] = big

        for g in range(_GROUPS):
            cols = jnp.int32(g * _LANES) + lane

            def merge_step(_, st):
                i, j, qprev, w = st
                av = plsc.load_gather(abuf, [i * _NC + cols])
                bv = plsc.load_gather(bbuf, [j * _NC + cols])
                take_a = av <= bv
                q = jnp.minimum(av, bv)
                d = (i - j).astype(jnp.float32)
                w = w + (q - qprev) * d * d
                one = jnp.ones((_LANES,), jnp.int32)
                zero = jnp.zeros((_LANES,), jnp.int32)
                i = i + jnp.where(take_a, one, zero)
                j = j + jnp.where(take_a, zero, one)
                return i, j, q, w

            init = (
                jnp.zeros((_LANES,), jnp.int32),
                jnp.zeros((_LANES,), jnp.int32),
                jnp.zeros((_LANES,), jnp.float32),
                jnp.zeros((_LANES,), jnp.float32),
            )
            _, _, _, w = lax.fori_loop(0, 2 * (_NT - 1), merge_step, init)

            valid = (totals[g] != 0.0) & (totals[_GROUPS + g] != 0.0)
            w_acc = w_acc + jnp.where(valid, w, jnp.zeros((_LANES,), jnp.float32))
        return w_acc

    w_acc = lax.fori_loop(0, _PPW, do_pair, jnp.zeros((_LANES,), jnp.float32))
    wstage[...] = w_acc * _CSQ
    pltpu.sync_copy(wstage, out_hbm.at[wid])


@jax.jit
def kernel(x, y):
    mesh = plsc.VectorSubcoreMesh(
        core_axis_name="c", subcore_axis_name="s",
        num_cores=_CORES, num_subcores=_SUBCORES,
    )
    run = functools.partial(
        pl.kernel,
        out_type=jax.ShapeDtypeStruct((_NW, _LANES), jnp.float32),
        mesh=mesh,
        compiler_params=pltpu.CompilerParams(needs_layout_passes=False),
        scratch_types=[
            pltpu.VMEM((_NT, _NC), jnp.float32),
            pltpu.VMEM((_NT, _NC), jnp.float32),
            pltpu.VMEM((_NT * _NC,), jnp.float32),
            pltpu.VMEM((_NT * _NC,), jnp.float32),
            pltpu.VMEM((_LANES,), jnp.float32),
        ],
    )(_tile_body)
    partials = run(x, y)
    return jnp.sum(partials)


# speedup vs baseline: 25799.0222x; 1.4196x over previous
import functools

import jax
import jax.numpy as jnp
from jax import lax
from jax.experimental import pallas as pl
from jax.experimental.pallas import tpu as pltpu
from jax.experimental.pallas import tpu_sc as plsc

_NB, _NT, _NR, _NC = 8, 128, 64, 64
_CORES, _SUBCORES, _LANES = 2, 16, 16
_NW = _CORES * _SUBCORES
_PAIRS = _NB * _NR
_PPW = _PAIRS // _NW
_GROUPS = _NC // _LANES
_BIG = jnp.float32(3.0e38)
_CSQ = jnp.float32(0.001) * jnp.float32(0.001)


def _tile_body(x_hbm, y_hbm, out_hbm, xbuf, ybuf, abuf, bbuf, wstage):
    wid = lax.axis_index("s") * _CORES + lax.axis_index("c")
    lane = lax.iota(jnp.int32, _LANES)

    def do_pair(p, w_acc):
        pair = wid * _PPW + p
        b = pair // _NR
        r = pair - b * _NR
        pltpu.sync_copy(x_hbm.at[b, :, r, :], xbuf)
        pltpu.sync_copy(y_hbm.at[b, :, r, :], ybuf)

        def cum_step(t, accs):
            out = []
            for g in range(_GROUPS):
                sl = pl.ds(g * _LANES, _LANES)
                fl = pl.ds(t * _NC + g * _LANES, _LANES)
                xa = xbuf[t, sl]
                aa = accs[g] + xa * xa
                abuf[fl] = aa
                ya = ybuf[t, sl]
                bb = accs[_GROUPS + g] + ya * ya
                bbuf[fl] = bb
                out.append(aa)
                out.append(bb)
            return tuple(out[0::2]) + tuple(out[1::2])

        zeros = tuple(jnp.zeros((_LANES,), jnp.float32) for _ in range(2 * _GROUPS))
        totals = lax.fori_loop(0, _NT, cum_step, zeros)

        big = jnp.full((_LANES,), _BIG, jnp.float32)
        for g in range(_GROUPS):
            fl = pl.ds((_NT - 1) * _NC + g * _LANES, _LANES)
            abuf[fl] = big
            bbuf[fl] = big

        cols = [jnp.int32(g * _LANES) + lane for g in range(_GROUPS)]
        one = jnp.ones((_LANES,), jnp.int32)
        zero = jnp.zeros((_LANES,), jnp.int32)

        def merge_step(_, st):
            iv, jv, qprev, wv = st
            iv2, jv2, q2, wv2 = [], [], [], []
            for g in range(_GROUPS):
                av = plsc.load_gather(abuf, [iv[g] * _NC + cols[g]])
                bv = plsc.load_gather(bbuf, [jv[g] * _NC + cols[g]])
                take_a = av <= bv
                q = jnp.minimum(av, bv)
                d = (iv[g] - jv[g]).astype(jnp.float32)
                wv2.append(wv[g] + (q - qprev[g]) * d * d)
                iv2.append(iv[g] + jnp.where(take_a, one, zero))
                jv2.append(jv[g] + jnp.where(take_a, zero, one))
                q2.append(q)
            return tuple(iv2), tuple(jv2), tuple(q2), tuple(wv2)

        izero = tuple(zero for _ in range(_GROUPS))
        fzero = tuple(jnp.zeros((_LANES,), jnp.float32) for _ in range(_GROUPS))
        _, _, _, wfin = lax.fori_loop(
            0, 2 * (_NT - 1), merge_step, (izero, izero, fzero, fzero)
        )

        for g in range(_GROUPS):
            valid = (totals[g] != 0.0) & (totals[_GROUPS + g] != 0.0)
            w_acc = w_acc + jnp.where(valid, wfin[g],
                                      jnp.zeros((_LANES,), jnp.float32))
        return w_acc

    w_acc = lax.fori_loop(0, _PPW, do_pair, jnp.zeros((_LANES,), jnp.float32))
    wstage[...] = w_acc * _CSQ
    pltpu.sync_copy(wstage, out_hbm.at[wid])


@jax.jit
def kernel(x, y):
    mesh = plsc.VectorSubcoreMesh(
        core_axis_name="c", subcore_axis_name="s",
        num_cores=_CORES, num_subcores=_SUBCORES,
    )
    run = functools.partial(
        pl.kernel,
        out_type=jax.ShapeDtypeStruct((_NW, _LANES), jnp.float32),
        mesh=mesh,
        compiler_params=pltpu.CompilerParams(needs_layout_passes=False),
        scratch_types=[
            pltpu.VMEM((_NT, _NC), jnp.float32),
            pltpu.VMEM((_NT, _NC), jnp.float32),
            pltpu.VMEM((_NT * _NC,), jnp.float32),
            pltpu.VMEM((_NT * _NC,), jnp.float32),
            pltpu.VMEM((_LANES,), jnp.float32),
        ],
    )(_tile_body)
    partials = run(x, y)
    return jnp.sum(partials)


# speedup vs baseline: 30398.5588x; 1.1783x over previous
import functools

import jax
import jax.numpy as jnp
from jax import lax
from jax.experimental import pallas as pl
from jax.experimental.pallas import tpu as pltpu
from jax.experimental.pallas import tpu_sc as plsc

_NB, _NT, _NR, _NC = 8, 128, 64, 64
_CORES, _SUBCORES, _LANES = 2, 16, 16
_NW = _CORES * _SUBCORES
_PAIRS = _NB * _NR
_PPW = _PAIRS // _NW
_GROUPS = _NC // _LANES
_BIG = jnp.float32(3.0e38)
_CSQ = jnp.float32(0.001) * jnp.float32(0.001)


def _tile_body(x_hbm, y_hbm, out_hbm, xbuf, ybuf, abuf, bbuf, wstage,
               semx, semy):
    wid = lax.axis_index("s") * _CORES + lax.axis_index("c")
    lane = lax.iota(jnp.int32, _LANES)

    def issue(p, parity):
        pair = wid * _PPW + p
        b = pair // _NR
        r = pair - b * _NR
        pltpu.async_copy(x_hbm.at[b, :, r, :], xbuf.at[parity], semx)
        pltpu.async_copy(y_hbm.at[b, :, r, :], ybuf.at[parity], semy)

    issue(0, 0)

    def do_pair(p, w_acc):
        parity = lax.rem(p, 2)
        pltpu.make_async_copy(x_hbm.at[0, :, 0, :], xbuf.at[parity], semx).wait()
        pltpu.make_async_copy(y_hbm.at[0, :, 0, :], ybuf.at[parity], semy).wait()

        @pl.when(p + 1 < _PPW)
        def _():
            issue(p + 1, 1 - parity)

        def cum_step(t, accs):
            out = []
            for g in range(_GROUPS):
                sl = pl.ds(g * _LANES, _LANES)
                fl = pl.ds(t * _NC + g * _LANES, _LANES)
                xa = xbuf[parity, t, sl]
                aa = accs[g] + xa * xa
                abuf[fl] = aa
                ya = ybuf[parity, t, sl]
                bb = accs[_GROUPS + g] + ya * ya
                bbuf[fl] = bb
                out.append(aa)
                out.append(bb)
            return tuple(out[0::2]) + tuple(out[1::2])

        zeros = tuple(jnp.zeros((_LANES,), jnp.float32) for _ in range(2 * _GROUPS))
        totals = lax.fori_loop(0, _NT, cum_step, zeros)

        big = jnp.full((_LANES,), _BIG, jnp.float32)
        for g in range(_GROUPS):
            fl = pl.ds((_NT - 1) * _NC + g * _LANES, _LANES)
            abuf[fl] = big
            bbuf[fl] = big

        cols = [jnp.int32(g * _LANES) + lane for g in range(_GROUPS)]
        one = jnp.ones((_LANES,), jnp.int32)
        zero = jnp.zeros((_LANES,), jnp.int32)

        def merge_step(_, st):
            iv, jv, qprev, wv = st
            iv2, jv2, q2, wv2 = [], [], [], []
            for g in range(_GROUPS):
                av = plsc.load_gather(abuf, [iv[g] * _NC + cols[g]])
                bv = plsc.load_gather(bbuf, [jv[g] * _NC + cols[g]])
                take_a = av <= bv
                q = jnp.minimum(av, bv)
                d = (iv[g] - jv[g]).astype(jnp.float32)
                wv2.append(wv[g] + (q - qprev[g]) * d * d)
                iv2.append(iv[g] + jnp.where(take_a, one, zero))
                jv2.append(jv[g] + jnp.where(take_a, zero, one))
                q2.append(q)
            return tuple(iv2), tuple(jv2), tuple(q2), tuple(wv2)

        izero = tuple(zero for _ in range(_GROUPS))
        fzero = tuple(jnp.zeros((_LANES,), jnp.float32) for _ in range(_GROUPS))
        _, _, _, wfin = lax.fori_loop(
            0, 2 * (_NT - 1), merge_step, (izero, izero, fzero, fzero)
        )

        for g in range(_GROUPS):
            valid = (totals[g] != 0.0) & (totals[_GROUPS + g] != 0.0)
            w_acc = w_acc + jnp.where(valid, wfin[g],
                                      jnp.zeros((_LANES,), jnp.float32))
        return w_acc

    w_acc = lax.fori_loop(0, _PPW, do_pair, jnp.zeros((_LANES,), jnp.float32))
    wstage[...] = w_acc * _CSQ
    pltpu.sync_copy(wstage, out_hbm.at[wid])


@jax.jit
def kernel(x, y):
    mesh = plsc.VectorSubcoreMesh(
        core_axis_name="c", subcore_axis_name="s",
        num_cores=_CORES, num_subcores=_SUBCORES,
    )
    run = functools.partial(
        pl.kernel,
        out_type=jax.ShapeDtypeStruct((_NW, _LANES), jnp.float32),
        mesh=mesh,
        compiler_params=pltpu.CompilerParams(needs_layout_passes=False),
        scratch_types=[
            pltpu.VMEM((2, _NT, _NC), jnp.float32),
            pltpu.VMEM((2, _NT, _NC), jnp.float32),
            pltpu.VMEM((_NT * _NC,), jnp.float32),
            pltpu.VMEM((_NT * _NC,), jnp.float32),
            pltpu.VMEM((_LANES,), jnp.float32),
            pltpu.SemaphoreType.DMA,
            pltpu.SemaphoreType.DMA,
        ],
    )(_tile_body)
    partials = run(x, y)
    return jnp.sum(partials)


# speedup vs baseline: 31581.4635x; 1.0389x over previous
import functools

import jax
import jax.numpy as jnp
from jax import lax
from jax.experimental import pallas as pl
from jax.experimental.pallas import tpu as pltpu
from jax.experimental.pallas import tpu_sc as plsc

_NB, _NT, _NR, _NC = 8, 128, 64, 64
_CORES, _SUBCORES, _LANES = 2, 16, 16
_NW = _CORES * _SUBCORES
_PAIRS = _NB * _NR
_PPW = _PAIRS // _NW
_GROUPS = _NC // _LANES
_BIG = jnp.float32(3.0e38)
_CSQ = jnp.float32(0.001) * jnp.float32(0.001)
_NROWS = _NT + 2
_CUM_UNROLL = 2
_MERGE_UNROLL = 4
_NSTEPS = 256


def _tile_body(x_hbm, y_hbm, out_hbm, xbuf, ybuf, abuf, bbuf, wstage,
               semx, semy):
    wid = lax.axis_index("s") * _CORES + lax.axis_index("c")
    lane = lax.iota(jnp.int32, _LANES)

    def issue(p, parity):
        pair = wid * _PPW + p
        b = pair // _NR
        r = pair - b * _NR
        pltpu.async_copy(x_hbm.at[b, :, r, :], xbuf.at[parity], semx)
        pltpu.async_copy(y_hbm.at[b, :, r, :], ybuf.at[parity], semy)

    issue(0, 0)

    def do_pair(p, w_acc):
        parity = lax.rem(p, 2)
        pltpu.make_async_copy(x_hbm.at[0, :, 0, :], xbuf.at[parity], semx).wait()
        pltpu.make_async_copy(y_hbm.at[0, :, 0, :], ybuf.at[parity], semy).wait()

        @pl.when(p + 1 < _PPW)
        def _():
            issue(p + 1, 1 - parity)

        def cum_step(tt, accs):
            for u in range(_CUM_UNROLL):
                t = tt * _CUM_UNROLL + u
                out = []
                for g in range(_GROUPS):
                    sl = pl.ds(g * _LANES, _LANES)
                    fl = pl.ds(t * _NC + g * _LANES, _LANES)
                    xa = xbuf[parity, t, sl]
                    aa = accs[g] + xa * xa
                    abuf[fl] = aa
                    ya = ybuf[parity, t, sl]
                    bb = accs[_GROUPS + g] + ya * ya
                    bbuf[fl] = bb
                    out.append(aa)
                    out.append(bb)
                accs = tuple(out[0::2]) + tuple(out[1::2])
            return accs

        zeros = tuple(jnp.zeros((_LANES,), jnp.float32) for _ in range(2 * _GROUPS))
        totals = lax.fori_loop(0, _NT // _CUM_UNROLL, cum_step, zeros)

        big = jnp.full((_LANES,), _BIG, jnp.float32)
        for g in range(_GROUPS):
            for row in (_NT - 1, _NT, _NT + 1):
                fl = pl.ds(row * _NC + g * _LANES, _LANES)
                abuf[fl] = big
                bbuf[fl] = big

        cols = [jnp.int32(g * _LANES) + lane for g in range(_GROUPS)]
        fone = jnp.float32(1.0)

        def one_step(g, st):
            ai, bi, av, bv, df, qprev, wv = st
            an = plsc.load_gather(abuf, [ai + _NC])
            bn = plsc.load_gather(bbuf, [bi + _NC])
            take_a = av <= bv
            q = jnp.minimum(av, bv)
            wv = wv + (q - qprev) * df * df
            ai = jnp.where(take_a, ai + _NC, ai)
            bi = jnp.where(take_a, bi, bi + _NC)
            df = df + jnp.where(take_a, fone, -fone)
            av = jnp.where(take_a, an, av)
            bv = jnp.where(take_a, bv, bn)
            return ai, bi, av, bv, df, q, wv

        def merge_step(_, sts):
            for u in range(_MERGE_UNROLL):
                sts = tuple(one_step(g, sts[g]) for g in range(_GROUPS))
            return sts

        init = []
        for g in range(_GROUPS):
            av0 = plsc.load_gather(abuf, [cols[g]])
            bv0 = plsc.load_gather(bbuf, [cols[g]])
            init.append((cols[g], cols[g], av0, bv0,
                         jnp.zeros((_LANES,), jnp.float32),
                         jnp.zeros((_LANES,), jnp.float32),
                         jnp.zeros((_LANES,), jnp.float32)))
        sts = lax.fori_loop(0, _NSTEPS // _MERGE_UNROLL, merge_step, tuple(init))

        for g in range(_GROUPS):
            valid = (totals[g] != 0.0) & (totals[_GROUPS + g] != 0.0)
            w_acc = w_acc + jnp.where(valid, sts[g][6],
                                      jnp.zeros((_LANES,), jnp.float32))
        return w_acc

    w_acc = lax.fori_loop(0, _PPW, do_pair, jnp.zeros((_LANES,), jnp.float32))
    wstage[...] = w_acc * _CSQ
    pltpu.sync_copy(wstage, out_hbm.at[wid])


@jax.jit
def kernel(x, y):
    mesh = plsc.VectorSubcoreMesh(
        core_axis_name="c", subcore_axis_name="s",
        num_cores=_CORES, num_subcores=_SUBCORES,
    )
    run = functools.partial(
        pl.kernel,
        out_type=jax.ShapeDtypeStruct((_NW, _LANES), jnp.float32),
        mesh=mesh,
        compiler_params=pltpu.CompilerParams(needs_layout_passes=False),
        scratch_types=[
            pltpu.VMEM((2, _NT, _NC), jnp.float32),
            pltpu.VMEM((2, _NT, _NC), jnp.float32),
            pltpu.VMEM((_NROWS * _NC,), jnp.float32),
            pltpu.VMEM((_NROWS * _NC,), jnp.float32),
            pltpu.VMEM((_LANES,), jnp.float32),
            pltpu.SemaphoreType.DMA,
            pltpu.SemaphoreType.DMA,
        ],
    )(_tile_body)
    partials = run(x, y)
    return jnp.sum(partials)


# speedup vs baseline: 31897.9068x; 1.0100x over previous
import functools

import jax
import jax.numpy as jnp
import numpy as np
from jax import lax
from jax.experimental import pallas as pl
from jax.experimental.pallas import tpu as pltpu
from jax.experimental.pallas import tpu_sc as plsc

_NB, _NT, _NR, _NC = 8, 128, 64, 64
_CORES, _SUBCORES, _LANES = 2, 16, 16
_NW = _CORES * _SUBCORES
_PAIRS = _NB * _NR
_PPW = _PAIRS // _NW
_GROUPS = _NC // _LANES
_BIG = float(np.float32(3.0e38))
_CSQ = float(np.float32(0.001) * np.float32(0.001))
_NROWS = _NT + 2
_CUM_UNROLL = 4
_MERGE_UNROLL = 8
_NSTEPS = 256


def _tile_body(x_hbm, y_hbm, out_hbm, xbuf, ybuf, abuf, bbuf, wstage,
               semx, semy):
    wid = lax.axis_index("s") * _CORES + lax.axis_index("c")
    lane = lax.iota(jnp.int32, _LANES)

    def issue(p, parity):
        pair = wid * _PPW + p
        b = pair // _NR
        r = pair - b * _NR
        pltpu.async_copy(x_hbm.at[b, :, r, :], xbuf.at[parity], semx)
        pltpu.async_copy(y_hbm.at[b, :, r, :], ybuf.at[parity], semy)

    issue(0, 0)

    def do_pair(p, w_acc):
        parity = lax.rem(p, 2)
        pltpu.make_async_copy(x_hbm.at[0, :, 0, :], xbuf.at[parity], semx).wait()
        pltpu.make_async_copy(y_hbm.at[0, :, 0, :], ybuf.at[parity], semy).wait()

        @pl.when(p + 1 < _PPW)
        def _():
            issue(p + 1, 1 - parity)

        def cum_step(tt, accs):
            for u in range(_CUM_UNROLL):
                t = tt * _CUM_UNROLL + u
                out = []
                for g in range(_GROUPS):
                    sl = pl.ds(g * _LANES, _LANES)
                    fl = pl.ds(t * _NC + g * _LANES, _LANES)
                    xa = xbuf[parity, t, sl]
                    aa = accs[g] + xa * xa
                    abuf[fl] = aa
                    ya = ybuf[parity, t, sl]
                    bb = accs[_GROUPS + g] + ya * ya
                    bbuf[fl] = bb
                    out.append(aa)
                    out.append(bb)
                accs = tuple(out[0::2]) + tuple(out[1::2])
            return accs

        zeros = tuple(jnp.zeros((_LANES,), jnp.float32) for _ in range(2 * _GROUPS))
        totals = lax.fori_loop(0, _NT // _CUM_UNROLL, cum_step, zeros)

        big = jnp.full((_LANES,), _BIG, jnp.float32)
        for g in range(_GROUPS):
            for row in (_NT - 1, _NT, _NT + 1):
                fl = pl.ds(row * _NC + g * _LANES, _LANES)
                abuf[fl] = big
                bbuf[fl] = big

        cols = [jnp.int32(g * _LANES) + lane for g in range(_GROUPS)]
        fone = jnp.float32(1.0)

        def one_step(g, st):
            ai, bi, av, bv, df, qprev, wv = st
            an = plsc.load_gather(abuf, [ai + _NC])
            bn = plsc.load_gather(bbuf, [bi + _NC])
            take_a = av <= bv
            q = jnp.minimum(av, bv)
            wv = wv + (q - qprev) * df * df
            ai = jnp.where(take_a, ai + _NC, ai)
            bi = jnp.where(take_a, bi, bi + _NC)
            df = df + jnp.where(take_a, fone, -fone)
            av = jnp.where(take_a, an, av)
            bv = jnp.where(take_a, bv, bn)
            return ai, bi, av, bv, df, q, wv

        def merge_step(_, sts):
            for u in range(_MERGE_UNROLL):
                sts = tuple(one_step(g, sts[g]) for g in range(_GROUPS))
            return sts

        init = []
        for g in range(_GROUPS):
            av0 = plsc.load_gather(abuf, [cols[g]])
            bv0 = plsc.load_gather(bbuf, [cols[g]])
            init.append((cols[g], cols[g], av0, bv0,
                         jnp.zeros((_LANES,), jnp.float32),
                         jnp.zeros((_LANES,), jnp.float32),
                         jnp.zeros((_LANES,), jnp.float32)))
        sts = lax.fori_loop(0, _NSTEPS // _MERGE_UNROLL, merge_step, tuple(init))

        for g in range(_GROUPS):
            valid = (totals[g] != 0.0) & (totals[_GROUPS + g] != 0.0)
            w_acc = w_acc + jnp.where(valid, sts[g][6],
                                      jnp.zeros((_LANES,), jnp.float32))
        return w_acc

    w_acc = lax.fori_loop(0, _PPW, do_pair, jnp.zeros((_LANES,), jnp.float32))
    wstage[...] = w_acc * _CSQ
    pltpu.sync_copy(wstage, out_hbm.at[wid])


@jax.jit
def kernel(x, y):
    mesh = plsc.VectorSubcoreMesh(
        core_axis_name="c", subcore_axis_name="s",
        num_cores=_CORES, num_subcores=_SUBCORES,
    )
    run = functools.partial(
        pl.kernel,
        out_type=jax.ShapeDtypeStruct((_NW, _LANES), jnp.float32),
        mesh=mesh,
        compiler_params=pltpu.CompilerParams(needs_layout_passes=False),
        scratch_types=[
            pltpu.VMEM((2, _NT, _NC), jnp.float32),
            pltpu.VMEM((2, _NT, _NC), jnp.float32),
            pltpu.VMEM((_NROWS * _NC,), jnp.float32),
            pltpu.VMEM((_NROWS * _NC,), jnp.float32),
            pltpu.VMEM((_LANES,), jnp.float32),
            pltpu.SemaphoreType.DMA,
            pltpu.SemaphoreType.DMA,
        ],
    )(_tile_body)
    partials = run(x, y)
    return jnp.sum(partials)
